# Initial kernel scaffold; baseline (speedup 1.0000x reference)
import jax
import jax.numpy as jnp
from jax.experimental import pallas as pl


def kernel(X, batch, W, b):
    raise NotImplementedError("write your pallas kernel here")


# speedup vs baseline: 12.9867x; 12.9867x over previous
import functools

import jax
import jax.numpy as jnp
from jax import lax
from jax.experimental import pallas as pl
from jax.experimental.pallas import tpu as pltpu
from jax.experimental.pallas import tpu_sc as plsc

N = 100000
DX = 256
DY = 256
S = 64

NC = 2
NS = 16
NW = NC * NS
ROWS_W = N // NW
CHUNK = 125
NCHUNK = ROWS_W // CHUNK

KV = 8
GW = 16 * KV
NG = DX // GW

OFF_PAD = 80


def _off_at(off_vmem, i):
  v = off_vmem[pl.ds(i, 16)]
  return v[0]


def _sc_body(x_hbm, off_hbm, out_hbm, xbuf0, xbuf1, offv, acc, sem0, sem1):
  wid = lax.axis_index("s") * NC + lax.axis_index("c")
  lo = wid * ROWS_W

  pltpu.sync_copy(off_hbm, offv)

  zeros = jnp.zeros((16,), jnp.float32)
  ninf = jnp.full((16,), -jnp.inf, jnp.float32)
  pinf = jnp.full((16,), jnp.inf, jnp.float32)

  def init_body(r, _):
    for k in range(DX // 16):
      acc[0, r, pl.ds(16 * k, 16)] = zeros
      acc[1, r, pl.ds(16 * k, 16)] = zeros
      acc[2, r, pl.ds(16 * k, 16)] = ninf
      acc[3, r, pl.ds(16 * k, 16)] = pinf
    return 0
  lax.fori_loop(0, S, init_body, 0)

  s0 = jnp.int32(0)

  def start_copy(c, buf, sem):
    base = (lo + c * CHUNK) * DX
    return pltpu.async_copy(x_hbm.at[pl.ds(base, CHUNK * DX)], buf, sem)

  def wait_copy(buf, sem):
    pltpu.make_async_copy(x_hbm.at[pl.ds(0, CHUNK * DX)], buf, sem).wait()

  def process_chunk(c, s, buf):
    base = lo + c * CHUNK

    def run_cond(carry):
      s_, row = carry
      return row < CHUNK

    def run_body(carry):
      s_, row = carry
      off_s1 = _off_at(offv, s_ + 1)
      e = jnp.minimum(off_s1 - base, CHUNK)

      @pl.when(e > row)
      def _():
        for g in range(NG):
          col0 = g * GW
          a_sum = [acc[0, s_, pl.ds(col0 + 16 * k, 16)] for k in range(KV)]
          a_sq = [acc[1, s_, pl.ds(col0 + 16 * k, 16)] for k in range(KV)]
          a_mx = [acc[2, s_, pl.ds(col0 + 16 * k, 16)] for k in range(KV)]
          a_mn = [acc[3, s_, pl.ds(col0 + 16 * k, 16)] for k in range(KV)]

          def row_body(r, carry_):
            sm, sq, mx, mn = carry_
            sm, sq, mx, mn = list(sm), list(sq), list(mx), list(mn)
            rbase = r * DX + col0
            for k in range(KV):
              v = buf[pl.ds(rbase + 16 * k, 16)]
              sm[k] = sm[k] + v
              sq[k] = sq[k] + v * v
              mx[k] = jnp.maximum(mx[k], v)
              mn[k] = jnp.minimum(mn[k], v)
            return tuple(sm), tuple(sq), tuple(mx), tuple(mn)

          a_sum, a_sq, a_mx, a_mn = lax.fori_loop(
              row, e, row_body, (tuple(a_sum), tuple(a_sq), tuple(a_mx),
                                 tuple(a_mn)))
          for k in range(KV):
            acc[0, s_, pl.ds(col0 + 16 * k, 16)] = a_sum[k]
            acc[1, s_, pl.ds(col0 + 16 * k, 16)] = a_sq[k]
            acc[2, s_, pl.ds(col0 + 16 * k, 16)] = a_mx[k]
            acc[3, s_, pl.ds(col0 + 16 * k, 16)] = a_mn[k]

      s_next = jnp.where(off_s1 <= base + CHUNK, s_ + 1, s_)
      row_next = jnp.maximum(row, e)
      return s_next, row_next

    s_out, _ = lax.while_loop(run_cond, run_body, (s, jnp.int32(0)))
    return s_out

  start_copy(0, xbuf0, sem0)
  start_copy(1, xbuf1, sem1)

  def chunk_pair(i2, s):
    c0 = 2 * i2
    wait_copy(xbuf0, sem0)
    s = process_chunk(c0, s, xbuf0)
    start_copy(c0 + 2, xbuf0, sem0)

    c1 = c0 + 1
    wait_copy(xbuf1, sem1)
    s = process_chunk(c1, s, xbuf1)

    @pl.when(c1 + 2 < NCHUNK)
    def _():
      start_copy(c1 + 2, xbuf1, sem1)
    return s

  s = lax.fori_loop(0, (NCHUNK - 1) // 2, chunk_pair, s0)

  wait_copy(xbuf0, sem0)
  process_chunk(NCHUNK - 1, s, xbuf0)

  pltpu.sync_copy(acc, out_hbm.at[wid])


_sc_partials = functools.partial(
    pl.kernel,
    out_type=jax.ShapeDtypeStruct((NW, 4, S, DX), jnp.float32),
    mesh=plsc.VectorSubcoreMesh(
        core_axis_name="c", subcore_axis_name="s", num_cores=NC,
        num_subcores=NS),
    scratch_types=[
        pltpu.VMEM((CHUNK * DX,), jnp.float32),
        pltpu.VMEM((CHUNK * DX,), jnp.float32),
        pltpu.VMEM((OFF_PAD,), jnp.int32),
        pltpu.VMEM((4, S, DX), jnp.float32),
        pltpu.SemaphoreType.DMA,
        pltpu.SemaphoreType.DMA,
    ],
    compiler_params=pltpu.CompilerParams(needs_layout_passes=False),
)(_sc_body)


def _tc_body(p_ref, off_ref, w_ref, b_ref, out_ref):
  p = p_ref[...]
  sums = jnp.sum(p[:, 0], axis=0)
  sqs = jnp.sum(p[:, 1], axis=0)
  maxs = jnp.max(p[:, 2], axis=0)
  mins = jnp.min(p[:, 3], axis=0)

  off = off_ref[0, :]
  counts = (off[1:S + 1] - off[:S]).astype(jnp.float32)
  denom = jnp.maximum(counts, 1.0)[:, None]
  m = sums / denom
  var = sqs / denom - m * m
  z = jnp.concatenate([m, mins, maxs, var], axis=1)
  out = lax.dot_general(z, w_ref[...], (((1,), (1,)), ((), ())),
                        preferred_element_type=jnp.float32)
  out_ref[...] = out + b_ref[...]


def kernel(X, batch, W, b):
  seg = batch.astype(jnp.int32)
  off = jnp.searchsorted(seg, jnp.arange(S + 1, dtype=jnp.int32),
                         side="left").astype(jnp.int32)
  off_pad = jnp.concatenate([off, jnp.full((OFF_PAD - S - 1,), N, jnp.int32)])

  partials = _sc_partials(X.reshape(-1), off_pad)

  out = pl.pallas_call(
      _tc_body,
      out_shape=jax.ShapeDtypeStruct((S, DY), jnp.float32),
  )(partials, off[None, :], W, b[None, :])
  return out


# speedup vs baseline: 18.2397x; 1.4045x over previous
import functools

import jax
import jax.numpy as jnp
from jax import lax
from jax.experimental import pallas as pl
from jax.experimental.pallas import tpu as pltpu
from jax.experimental.pallas import tpu_sc as plsc

N = 100000
DX = 256
DY = 256
S = 64

NC = 2
NS = 16
NW = NC * NS
ROWS_W = 3128
CHUNK = 120
NCHUNK = -(-ROWS_W // CHUNK)

KV = 8
GW = 16 * KV
NG = DX // GW

OFF_PAD = 80


def _off_at(off_vmem, i):
  v = off_vmem[pl.ds(i, 16)]
  return v[0]


def _sc_body(x_hbm, off_hbm, out_hbm, xbuf0, xbuf1, offv, acc, sem0, sem1):
  wid = lax.axis_index("s") * NC + lax.axis_index("c")
  lo = wid * ROWS_W
  hi = jnp.minimum(lo + ROWS_W, N)

  def chunk_base(c):
    nominal = lo + c * CHUNK
    base = jnp.minimum(nominal, hi - CHUNK)
    return base, nominal - base

  pltpu.sync_copy(off_hbm, offv)

  zeros = jnp.zeros((16,), jnp.float32)
  ninf = jnp.full((16,), -jnp.inf, jnp.float32)
  pinf = jnp.full((16,), jnp.inf, jnp.float32)

  def init_body(r, _):
    for k in range(DX // 16):
      acc[0, r, pl.ds(16 * k, 16)] = zeros
      acc[1, r, pl.ds(16 * k, 16)] = zeros
      acc[2, r, pl.ds(16 * k, 16)] = ninf
      acc[3, r, pl.ds(16 * k, 16)] = pinf
    return 0
  lax.fori_loop(0, S, init_body, 0)

  s0 = jnp.int32(0)

  def start_copy(c, buf, sem):
    base, _ = chunk_base(c)
    return pltpu.async_copy(x_hbm.at[pl.ds(base, CHUNK)], buf, sem)

  def wait_copy(buf, sem):
    pltpu.make_async_copy(x_hbm.at[pl.ds(0, CHUNK)], buf, sem).wait()

  def process_chunk(c, s, buf):
    base, row0 = chunk_base(c)

    def run_cond(carry):
      s_, row = carry
      return row < CHUNK

    def run_body(carry):
      s_, row = carry
      off_s1 = _off_at(offv, s_ + 1)
      e = jnp.minimum(off_s1 - base, CHUNK)

      @pl.when(e > row)
      def _():
        for g in range(NG):
          col0 = g * GW
          a_sum = [acc[0, s_, pl.ds(col0 + 16 * k, 16)] for k in range(KV)]
          a_sq = [acc[1, s_, pl.ds(col0 + 16 * k, 16)] for k in range(KV)]
          a_mx = [acc[2, s_, pl.ds(col0 + 16 * k, 16)] for k in range(KV)]
          a_mn = [acc[3, s_, pl.ds(col0 + 16 * k, 16)] for k in range(KV)]

          def row_body(r, carry_):
            sm, sq, mx, mn = carry_
            sm, sq, mx, mn = list(sm), list(sq), list(mx), list(mn)
            for k in range(KV):
              v = buf[r, pl.ds(col0 + 16 * k, 16)]
              sm[k] = sm[k] + v
              sq[k] = sq[k] + v * v
              mx[k] = jnp.maximum(mx[k], v)
              mn[k] = jnp.minimum(mn[k], v)
            return tuple(sm), tuple(sq), tuple(mx), tuple(mn)

          a_sum, a_sq, a_mx, a_mn = lax.fori_loop(
              row, e, row_body, (tuple(a_sum), tuple(a_sq), tuple(a_mx),
                                 tuple(a_mn)))
          for k in range(KV):
            acc[0, s_, pl.ds(col0 + 16 * k, 16)] = a_sum[k]
            acc[1, s_, pl.ds(col0 + 16 * k, 16)] = a_sq[k]
            acc[2, s_, pl.ds(col0 + 16 * k, 16)] = a_mx[k]
            acc[3, s_, pl.ds(col0 + 16 * k, 16)] = a_mn[k]

      s_next = jnp.where(off_s1 <= base + CHUNK, s_ + 1, s_)
      row_next = jnp.maximum(row, e)
      return s_next, row_next

    s_out, _ = lax.while_loop(run_cond, run_body, (s, row0))
    return s_out

  start_copy(0, xbuf0, sem0)
  start_copy(1, xbuf1, sem1)

  def chunk_pair(i2, s):
    c0 = 2 * i2
    wait_copy(xbuf0, sem0)
    s = process_chunk(c0, s, xbuf0)
    start_copy(c0 + 2, xbuf0, sem0)

    c1 = c0 + 1
    wait_copy(xbuf1, sem1)
    s = process_chunk(c1, s, xbuf1)

    @pl.when(c1 + 2 < NCHUNK)
    def _():
      start_copy(c1 + 2, xbuf1, sem1)
    return s

  s = lax.fori_loop(0, (NCHUNK - 1) // 2, chunk_pair, s0)

  wait_copy(xbuf0, sem0)
  process_chunk(NCHUNK - 1, s, xbuf0)

  pltpu.sync_copy(acc, out_hbm.at[wid])


_sc_partials = functools.partial(
    pl.kernel,
    out_type=jax.ShapeDtypeStruct((NW, 4, S, DX), jnp.float32),
    mesh=plsc.VectorSubcoreMesh(
        core_axis_name="c", subcore_axis_name="s", num_cores=NC,
        num_subcores=NS),
    scratch_types=[
        pltpu.VMEM((CHUNK, DX), jnp.float32),
        pltpu.VMEM((CHUNK, DX), jnp.float32),
        pltpu.VMEM((OFF_PAD,), jnp.int32),
        pltpu.VMEM((4, S, DX), jnp.float32),
        pltpu.SemaphoreType.DMA,
        pltpu.SemaphoreType.DMA,
    ],
    compiler_params=pltpu.CompilerParams(needs_layout_passes=False),
)(_sc_body)


def _tc_body(p_ref, off_ref, w_ref, b_ref, out_ref):
  p = p_ref[...]
  sums = jnp.sum(p[:, 0], axis=0)
  sqs = jnp.sum(p[:, 1], axis=0)
  maxs = jnp.max(p[:, 2], axis=0)
  mins = jnp.min(p[:, 3], axis=0)

  off = off_ref[0, :]
  counts = (off[1:S + 1] - off[:S]).astype(jnp.float32)
  denom = jnp.maximum(counts, 1.0)[:, None]
  m = sums / denom
  var = sqs / denom - m * m
  z = jnp.concatenate([m, mins, maxs, var], axis=1)
  out = lax.dot_general(z, w_ref[...], (((1,), (1,)), ((), ())),
                        preferred_element_type=jnp.float32)
  out_ref[...] = out + b_ref[...]


def kernel(X, batch, W, b):
  seg = batch.astype(jnp.int32)
  off = jnp.searchsorted(seg, jnp.arange(S + 1, dtype=jnp.int32),
                         side="left").astype(jnp.int32)
  off_pad = jnp.concatenate([off, jnp.full((OFF_PAD - S - 1,), N, jnp.int32)])

  partials = _sc_partials(X, off_pad)

  out = pl.pallas_call(
      _tc_body,
      out_shape=jax.ShapeDtypeStruct((S, DY), jnp.float32),
  )(partials, off[None, :], W, b[None, :])
  return out


# speedup vs baseline: 23.0987x; 1.2664x over previous
import functools

import jax
import jax.numpy as jnp
from jax import lax
from jax.experimental import pallas as pl
from jax.experimental.pallas import tpu as pltpu
from jax.experimental.pallas import tpu_sc as plsc

N = 100000
DX = 256
DY = 256
S = 64

NC = 2
NS = 16
NW = NC * NS
ROWS_W = 3128
N_PAD = NW * ROWS_W
CHUNK = 112
NCHUNK = -(-ROWS_W // CHUNK)

KV = 8
GW = 16 * KV
NG = DX // GW

BPAD = 16


def _sc_body(x_hbm, b_hbm, out_hbm, cnt_hbm, xbuf0, xbuf1, bbuf, acc, cntv,
             sem0, sem1):
  wid = lax.axis_index("s") * NC + lax.axis_index("c")
  lo = wid * ROWS_W
  hi = jnp.minimum(lo + ROWS_W, N)

  pltpu.sync_copy(b_hbm.at[pl.ds(lo, ROWS_W)], bbuf.at[pl.ds(0, ROWS_W)])

  def chunk_base(c):
    nominal = lo + c * CHUNK
    base = jnp.minimum(nominal, hi - CHUNK)
    return base, nominal - base

  zeros = jnp.zeros((16,), jnp.float32)
  ninf = jnp.full((16,), -jnp.inf, jnp.float32)
  pinf = jnp.full((16,), jnp.inf, jnp.float32)
  izeros = jnp.zeros((16,), jnp.int32)

  def init_body(r, _):
    for k in range(DX // 16):
      acc[0, r, pl.ds(16 * k, 16)] = zeros
      acc[1, r, pl.ds(16 * k, 16)] = zeros
      acc[2, r, pl.ds(16 * k, 16)] = ninf
      acc[3, r, pl.ds(16 * k, 16)] = pinf
    return 0
  lax.fori_loop(0, S, init_body, 0)
  for k in range((S + BPAD) // 16):
    cntv[0, pl.ds(16 * k, 16)] = izeros

  lane0 = jnp.arange(16, dtype=jnp.int32) == 0

  def start_copy(c, buf, sem):
    base, _ = chunk_base(c)
    return pltpu.async_copy(x_hbm.at[pl.ds(base, CHUNK)], buf, sem)

  def wait_copy(buf, sem):
    pltpu.make_async_copy(x_hbm.at[pl.ds(0, CHUNK)], buf, sem).wait()

  def process_chunk(c, buf):
    base, row0 = chunk_base(c)
    boff = base - lo

    def run_cond(row):
      return row < CHUNK

    def run_body(row):
      ids = bbuf[pl.ds(boff + row, 16)]
      s = ids[0]

      def scan_cond(carry):
        j, f = carry
        return (f >= 16) & (j < CHUNK)

      def scan_body(carry):
        j, _ = carry
        blk = bbuf[pl.ds(boff + j, 16)]
        f2 = plsc.all_reduce_ffs(blk != s)[0]
        return j + 16, f2

      j_end, f = lax.while_loop(scan_cond, scan_body, (row, jnp.int32(16)))
      e = jnp.minimum(jnp.where(f < 16, j_end - 16 + f, j_end),
                      jnp.int32(CHUNK))

      for g in range(NG):
        col0 = g * GW
        a_sum = tuple(acc[0, s, pl.ds(col0 + 16 * k, 16)] for k in range(KV))
        a_sq = tuple(acc[1, s, pl.ds(col0 + 16 * k, 16)] for k in range(KV))
        a_mx = tuple(acc[2, s, pl.ds(col0 + 16 * k, 16)] for k in range(KV))
        a_mn = tuple(acc[3, s, pl.ds(col0 + 16 * k, 16)] for k in range(KV))

        def row_body(r, carry_):
          sm, sq, mx, mn = carry_
          sm, sq, mx, mn = list(sm), list(sq), list(mx), list(mn)
          for k in range(KV):
            v = buf[r, pl.ds(col0 + 16 * k, 16)]
            sm[k] = sm[k] + v
            sq[k] = sq[k] + v * v
            mx[k] = jnp.maximum(mx[k], v)
            mn[k] = jnp.minimum(mn[k], v)
          return tuple(sm), tuple(sq), tuple(mx), tuple(mn)

        a_sum, a_sq, a_mx, a_mn = lax.fori_loop(
            row, e, row_body, (a_sum, a_sq, a_mx, a_mn))
        for k in range(KV):
          acc[0, s, pl.ds(col0 + 16 * k, 16)] = a_sum[k]
          acc[1, s, pl.ds(col0 + 16 * k, 16)] = a_sq[k]
          acc[2, s, pl.ds(col0 + 16 * k, 16)] = a_mx[k]
          acc[3, s, pl.ds(col0 + 16 * k, 16)] = a_mn[k]

      cv = cntv[0, pl.ds(s, 16)]
      cntv[0, pl.ds(s, 16)] = cv + jnp.where(lane0, e - row, jnp.int32(0))
      return e

    lax.while_loop(run_cond, run_body, row0)

  start_copy(0, xbuf0, sem0)
  start_copy(1, xbuf1, sem1)

  def chunk_pair(i2, carry):
    c0 = 2 * i2
    wait_copy(xbuf0, sem0)
    process_chunk(c0, xbuf0)

    @pl.when(c0 + 2 < NCHUNK)
    def _():
      start_copy(c0 + 2, xbuf0, sem0)

    c1 = c0 + 1
    wait_copy(xbuf1, sem1)
    process_chunk(c1, xbuf1)

    @pl.when(c1 + 2 < NCHUNK)
    def _():
      start_copy(c1 + 2, xbuf1, sem1)
    return carry

  lax.fori_loop(0, NCHUNK // 2, chunk_pair, 0)

  pltpu.sync_copy(acc, out_hbm.at[wid])
  pltpu.sync_copy(cntv, cnt_hbm.at[wid])


_sc_partials = functools.partial(
    pl.kernel,
    out_type=(
        jax.ShapeDtypeStruct((NW, 4, S, DX), jnp.float32),
        jax.ShapeDtypeStruct((NW, 8, S + BPAD), jnp.int32),
    ),
    mesh=plsc.VectorSubcoreMesh(
        core_axis_name="c", subcore_axis_name="s", num_cores=NC,
        num_subcores=NS),
    scratch_types=[
        pltpu.VMEM((CHUNK, DX), jnp.float32),
        pltpu.VMEM((CHUNK, DX), jnp.float32),
        pltpu.VMEM((ROWS_W + BPAD,), jnp.int32),
        pltpu.VMEM((4, S, DX), jnp.float32),
        pltpu.VMEM((8, S + BPAD), jnp.int32),
        pltpu.SemaphoreType.DMA,
        pltpu.SemaphoreType.DMA,
    ],
    compiler_params=pltpu.CompilerParams(needs_layout_passes=False),
)(_sc_body)


def _tc_body(p_ref, cnt_ref, w_ref, b_ref, out_ref):
  p = p_ref[...]
  sums = jnp.sum(p[:, 0], axis=0)
  sqs = jnp.sum(p[:, 1], axis=0)
  maxs = jnp.max(p[:, 2], axis=0)
  mins = jnp.min(p[:, 3], axis=0)

  counts = jnp.sum(cnt_ref[...][:, 0, :S], axis=0).astype(jnp.float32)
  denom = jnp.maximum(counts, 1.0)[:, None]
  m = sums / denom
  var = sqs / denom - m * m
  z = jnp.concatenate([m, mins, maxs, var], axis=1)
  out = lax.dot_general(z, w_ref[...], (((1,), (1,)), ((), ())),
                        preferred_element_type=jnp.float32)
  out_ref[...] = out + b_ref[...]


def kernel(X, batch, W, b):
  seg = batch.astype(jnp.int32)
  seg_pad = jnp.concatenate(
      [seg, jnp.full((N_PAD - N,), jnp.int32(1 << 30))])

  partials, cnts = _sc_partials(X, seg_pad)

  out = pl.pallas_call(
      _tc_body,
      out_shape=jax.ShapeDtypeStruct((S, DY), jnp.float32),
  )(partials, cnts, W, b[None, :])
  return out
